# baseline (speedup 1.0000x reference)
import functools

import jax
import jax.numpy as jnp
from jax import lax
from jax.experimental import pallas as pl
from jax.experimental.pallas import tpu as pltpu
from jax.experimental.pallas import tpu_sc as plsc

N = 16384
C = 1000
D = 128
CP = 1024

RB = 512
NBLK = N // RB

NT = 16
RW = N // NT
KCH = RW // 128
CSTR = CP // NT
FCHUNK = 256


def _argmax_body(x_ref, lab_ref, cnt_ref):
    x = x_ref[...]
    m = jnp.max(x, axis=1, keepdims=True)
    iota = lax.broadcasted_iota(jnp.int32, x.shape, 1)
    big = jnp.int32(2 ** 30)
    idx = jnp.min(jnp.where(x == m, iota, big), axis=1)
    lab_ref[...] = idx.reshape(1, 1, RB)

    iota_cp = lax.broadcasted_iota(jnp.int32, (CP, RB), 0)
    onehot_t = (iota_cp == idx[None, :]).astype(jnp.float32)
    ones16 = jnp.ones((RB, 16), jnp.float32)
    blkcnt = lax.dot_general(
        onehot_t, ones16, (((1,), (0,)), ((), ())),
        preferred_element_type=jnp.float32)

    @pl.when(pl.program_id(0) == 0)
    def _():
        cnt_ref[...] = jnp.zeros((CP, 16), jnp.float32)
    cnt_ref[...] += blkcnt


def _segment_loss_kernel():
    mesh = plsc.VectorSubcoreMesh(
        core_axis_name="c", subcore_axis_name="s", num_cores=1)

    @functools.partial(
        pl.kernel,
        out_type=[
            jax.ShapeDtypeStruct((16,), jnp.float32),
            jax.ShapeDtypeStruct((NT, 2, 16), jnp.float32),
        ],
        mesh=mesh,
        scratch_types=[
            pltpu.VMEM((KCH, 128), jnp.int32),
            pltpu.VMEM((FCHUNK, D), jnp.float32),
            pltpu.VMEM((CSTR, D), jnp.float32),
            pltpu.VMEM((CSTR, 16), jnp.float32),
            pltpu.VMEM((CSTR, D), jnp.float32),
            pltpu.VMEM((2, 16), jnp.float32),
            pltpu.VMEM((NT, 2, 16), jnp.float32),
            pltpu.VMEM((16,), jnp.float32),
            pltpu.VMEM_SHARED((CP, D), jnp.float32),
        ],
    )
    def seg(labels_hbm, feats_hbm, counts_hbm, text_hbm, out_hbm, parts_hbm,
            lab_v, feats_v, zbuf, cnt_v, t_v, pbuf, pv, obuf,
            sums_sh):
        sid = lax.axis_index("s")

        zf16 = jnp.zeros((16,), jnp.float32)

        def zrow(r, _):
            for j in range(D // 16):
                zbuf[r, pl.ds(j * 16, 16)] = zf16
            return 0
        lax.fori_loop(0, CSTR, zrow, 0)

        pltpu.sync_copy(zbuf, sums_sh.at[pl.ds(sid * CSTR, CSTR)])
        pltpu.sync_copy(labels_hbm.at[sid], lab_v)

        plsc.subcore_barrier()

        for t in range(RW // FCHUNK):
            pltpu.sync_copy(
                feats_hbm.at[pl.ds(sid * RW + t * FCHUNK, FCHUNK)], feats_v)
            for j in range(FCHUNK // 128):
                ch = t * (FCHUNK // 128) + j
                pltpu.sync_copy(feats_v.at[pl.ds(j * 128, 128)],
                                sums_sh.at[lab_v.at[ch]], add=True)

        plsc.subcore_barrier()

        pltpu.sync_copy(sums_sh.at[pl.ds(sid * CSTR, CSTR)], zbuf)
        pltpu.sync_copy(counts_hbm.at[pl.ds(sid * CSTR, CSTR)], cnt_v)
        pltpu.sync_copy(text_hbm.at[pl.ds(sid * CSTR, CSTR)], t_v)

        def cbody(c, carry):
            vt, nv = carry
            acc = jnp.zeros((16,), jnp.float32)
            for j in range(D // 16):
                acc = acc + zbuf[c, pl.ds(j * 16, 16)] * t_v[c, pl.ds(j * 16, 16)]
            cnt = cnt_v[c, :]
            mask = cnt > 0.0
            vt = vt + jnp.where(mask, acc / jnp.maximum(cnt, 1.0), 0.0)
            nv = nv + jnp.where(mask, 1.0, 0.0)
            return vt, nv

        vt, nv = lax.fori_loop(
            0, CSTR, cbody,
            (jnp.zeros((16,), jnp.float32), jnp.zeros((16,), jnp.float32)))

        pbuf[0, :] = vt
        pbuf[1, :] = nv
        pltpu.sync_copy(pbuf, parts_hbm.at[sid])

        plsc.subcore_barrier()

        @pl.when(sid == 0)
        def _():
            pltpu.sync_copy(parts_hbm, pv)
            lsum = jnp.zeros((16,), jnp.float32)
            nsum = jnp.zeros((16,), jnp.float32)
            for i in range(NT):
                lsum = lsum + pv[i, 0, :]
                nsum = nsum + pv[i, 1, :]
            num = jnp.float32(0.0)
            den = jnp.float32(0.0)
            for i in range(16):
                num = num + lsum[i]
                den = den + nsum[i]
            numv = jnp.full((16,), num, jnp.float32)
            denv = jnp.full((16,), den, jnp.float32) / 16.0
            obuf[:] = numv / denv
            pltpu.sync_copy(obuf, out_hbm)

    return seg


def kernel(logits, img_feats, text_norm_feats):
    labels3, counts = pl.pallas_call(
        _argmax_body,
        grid=(NBLK,),
        in_specs=[pl.BlockSpec((RB, C), lambda i: (i, 0))],
        out_specs=[
            pl.BlockSpec((1, 1, RB), lambda i: (i, 0, 0)),
            pl.BlockSpec((CP, 16), lambda i: (0, 0)),
        ],
        out_shape=[
            jax.ShapeDtypeStruct((NBLK, 1, RB), jnp.int32),
            jax.ShapeDtypeStruct((CP, 16), jnp.float32),
        ],
    )(logits)

    labels = labels3.reshape(NT, KCH, 128)

    text_pad = jnp.zeros((CP, D), jnp.float32).at[:C].set(text_norm_feats)

    out, _ = _segment_loss_kernel()(labels, img_feats, counts, text_pad)
    return out[0]


# speedup vs baseline: 1.8183x; 1.8183x over previous
import functools

import jax
import jax.numpy as jnp
from jax import lax
from jax.experimental import pallas as pl
from jax.experimental.pallas import tpu as pltpu
from jax.experimental.pallas import tpu_sc as plsc

N = 16384
C = 1000
D = 128
CP = 1024

RB = 512
NBLK = N // RB

NT = 16
RW = N // NT
KCH = RW // 128
CSTR = CP // NT
FCHUNK = 256


def _argmax_body(x_ref, lab_ref, cnt_ref):
    x = x_ref[...]
    m = jnp.max(x, axis=0, keepdims=True)
    iota = lax.broadcasted_iota(jnp.int32, x.shape, 0)
    big = jnp.int32(2 ** 30)
    idx = jnp.min(jnp.where(x == m, iota, big), axis=0)
    lab_ref[...] = idx.reshape(1, 1, RB)

    iota_cp = lax.broadcasted_iota(jnp.int32, (CP, RB), 0)
    onehot_t = (iota_cp == idx[None, :]).astype(jnp.float32)
    ones16 = jnp.ones((RB, 16), jnp.float32)
    blkcnt = lax.dot_general(
        onehot_t, ones16, (((1,), (0,)), ((), ())),
        preferred_element_type=jnp.float32)

    @pl.when(pl.program_id(0) == 0)
    def _():
        cnt_ref[...] = jnp.zeros((CP, 16), jnp.float32)
    cnt_ref[...] += blkcnt


def _segment_loss_kernel():
    mesh = plsc.VectorSubcoreMesh(
        core_axis_name="c", subcore_axis_name="s", num_cores=1)

    @functools.partial(
        pl.kernel,
        out_type=[
            jax.ShapeDtypeStruct((16,), jnp.float32),
            jax.ShapeDtypeStruct((NT, 2, 16), jnp.float32),
        ],
        mesh=mesh,
        scratch_types=[
            pltpu.VMEM((KCH, 128), jnp.int32),
            pltpu.VMEM((FCHUNK, D), jnp.float32),
            pltpu.VMEM((CSTR, D), jnp.float32),
            pltpu.VMEM((CSTR, 16), jnp.float32),
            pltpu.VMEM((CSTR, D), jnp.float32),
            pltpu.VMEM((2, 16), jnp.float32),
            pltpu.VMEM((NT, 2, 16), jnp.float32),
            pltpu.VMEM((16,), jnp.float32),
            pltpu.VMEM_SHARED((CP, D), jnp.float32),
        ],
    )
    def seg(labels_hbm, feats_hbm, counts_hbm, text_hbm, out_hbm, parts_hbm,
            lab_v, feats_v, zbuf, cnt_v, t_v, pbuf, pv, obuf,
            sums_sh):
        sid = lax.axis_index("s")

        zf16 = jnp.zeros((16,), jnp.float32)

        def zrow(r, _):
            for j in range(D // 16):
                zbuf[r, pl.ds(j * 16, 16)] = zf16
            return 0
        lax.fori_loop(0, CSTR, zrow, 0)

        pltpu.sync_copy(zbuf, sums_sh.at[pl.ds(sid * CSTR, CSTR)])
        pltpu.sync_copy(labels_hbm.at[sid], lab_v)

        plsc.subcore_barrier()

        for t in range(RW // FCHUNK):
            pltpu.sync_copy(
                feats_hbm.at[pl.ds(sid * RW + t * FCHUNK, FCHUNK)], feats_v)
            for j in range(FCHUNK // 128):
                ch = t * (FCHUNK // 128) + j
                pltpu.sync_copy(feats_v.at[pl.ds(j * 128, 128)],
                                sums_sh.at[lab_v.at[ch]], add=True)

        plsc.subcore_barrier()

        pltpu.sync_copy(sums_sh.at[pl.ds(sid * CSTR, CSTR)], zbuf)
        pltpu.sync_copy(counts_hbm.at[pl.ds(sid * CSTR, CSTR)], cnt_v)
        pltpu.sync_copy(text_hbm.at[pl.ds(sid * CSTR, CSTR)], t_v)

        def cbody(c, carry):
            vt, nv = carry
            acc = jnp.zeros((16,), jnp.float32)
            for j in range(D // 16):
                acc = acc + zbuf[c, pl.ds(j * 16, 16)] * t_v[c, pl.ds(j * 16, 16)]
            cnt = cnt_v[c, :]
            mask = cnt > 0.0
            vt = vt + jnp.where(mask, acc / jnp.maximum(cnt, 1.0), 0.0)
            nv = nv + jnp.where(mask, 1.0, 0.0)
            return vt, nv

        vt, nv = lax.fori_loop(
            0, CSTR, cbody,
            (jnp.zeros((16,), jnp.float32), jnp.zeros((16,), jnp.float32)))

        pbuf[0, :] = vt
        pbuf[1, :] = nv
        pltpu.sync_copy(pbuf, parts_hbm.at[sid])

        plsc.subcore_barrier()

        @pl.when(sid == 0)
        def _():
            pltpu.sync_copy(parts_hbm, pv)
            lsum = jnp.zeros((16,), jnp.float32)
            nsum = jnp.zeros((16,), jnp.float32)
            for i in range(NT):
                lsum = lsum + pv[i, 0, :]
                nsum = nsum + pv[i, 1, :]
            num = jnp.float32(0.0)
            den = jnp.float32(0.0)
            for i in range(16):
                num = num + lsum[i]
                den = den + nsum[i]
            numv = jnp.full((16,), num, jnp.float32)
            denv = jnp.full((16,), den, jnp.float32) / 16.0
            obuf[:] = numv / denv
            pltpu.sync_copy(obuf, out_hbm)

    return seg


def kernel(logits, img_feats, text_norm_feats):
    labels3, counts = pl.pallas_call(
        _argmax_body,
        grid=(NBLK,),
        in_specs=[pl.BlockSpec((C, RB), lambda i: (0, i))],
        out_specs=[
            pl.BlockSpec((1, 1, RB), lambda i: (i, 0, 0)),
            pl.BlockSpec((CP, 16), lambda i: (0, 0)),
        ],
        out_shape=[
            jax.ShapeDtypeStruct((NBLK, 1, RB), jnp.int32),
            jax.ShapeDtypeStruct((CP, 16), jnp.float32),
        ],
    )(logits.T)

    labels = labels3.reshape(NT, KCH, 128)

    text_pad = jnp.zeros((CP, D), jnp.float32).at[:C].set(text_norm_feats)

    out, _ = _segment_loss_kernel()(labels, img_feats, counts, text_pad)
    return out[0]
